# baseline (speedup 1.0000x reference)
import functools

import jax
import jax.numpy as jnp
from jax import lax
from jax.experimental import pallas as pl
from jax.experimental.pallas import tpu as pltpu


def _meu_kernel(W, HW, inv_hw, C,
                low_ref, hd_ref,
                wlm_ref, wl0_ref, wlp_ref, bl_ref,
                whm_ref, wh0_ref, whp_ref, bh_ref,
                cw1_ref, cb1_ref, cw2_ref, cb2_ref,
                pb2m_ref, pw2_ref, pb2_ref,
                o_ref):
    padw = low_ref.shape[2]
    padl = 2 * W

    colm = (lax.broadcasted_iota(jnp.int32, (1, padw), 1) - padl) % W
    mL = (colm != W - 1).astype(jnp.bfloat16)
    mR = (colm != 0).astype(jnp.bfloat16)

    def group_stack(v, vl, vr, ox):
        src = vl if ox == -1 else (vr if ox == 1 else v)
        return jnp.concatenate(
            [src[:, padl + oy * W + ox:padl + oy * W + ox + HW]
             for oy in (-1, 0, 1)], axis=0)

    def conv9(v, w_refs):
        vl = v * mL
        vr = v * mR
        return [jnp.dot(w_ref[...], group_stack(v, vl, vr, ox),
                        preferred_element_type=jnp.float32)
                for ox, w_ref in zip((-1, 0, 1), w_refs)]

    e0, e1, e2 = conv9(hd_ref[0], (whm_ref, wh0_ref, whp_ref))
    high = (e0 + e1) + (e2 + bh_ref[...])

    d0, d1, d2 = conv9(low_ref[0], (wlm_ref, wl0_ref, wlp_ref))
    lw = (d0 + d1) + d2
    low = lw[:C] + bl_ref[...]

    mean = ((jnp.sum(e0, axis=1, keepdims=True)
             + jnp.sum(e1, axis=1, keepdims=True)
             + jnp.sum(e2, axis=1, keepdims=True)) * inv_hw
            + bh_ref[...])
    h1 = jnp.dot(cw1_ref[...], mean, preferred_element_type=jnp.float32)
    h1 = jnp.maximum(h1 + cb1_ref[...], 0.0)
    ca = jax.nn.sigmoid(
        jnp.dot(cw2_ref[...], h1, preferred_element_type=jnp.float32)
        + cb2_ref[...])

    h2 = jnp.maximum(lw[C:] + pb2m_ref[...], 0.0)
    pa = jax.nn.sigmoid(
        jnp.sum(h2 * pw2_ref[...], axis=0, keepdims=True) + pb2_ref[...])

    o_ref[0] = (low * ca + high * pa).astype(o_ref.dtype)


def kernel(low_level, high_level, low_w, low_b, high_w, high_b,
           ca_w1, ca_b1, ca_w2, ca_b2, pa_w1, pa_b1, pa_w2, pa_b2):
    N, C, H, W = low_level.shape
    Ch, H2, W2 = high_level.shape[1], high_level.shape[2], high_level.shape[3]
    HW = H * W
    Cmid = pa_w1.shape[0]
    padw = HW + 4 * W

    low_in = jnp.pad(low_level.astype(jnp.bfloat16),
                     ((0, 0), (0, 0), (2, 2), (0, 0))).reshape(N, C, padw)

    hb = high_level.astype(jnp.bfloat16)
    cd = jnp.stack([hb, jnp.zeros_like(hb)], axis=-1).reshape(N, Ch, H2, W)
    hdc = jnp.concatenate([cd, jnp.zeros_like(cd)], axis=3)
    hd_in = jnp.pad(hdc, ((0, 0), (0, 0), (1, 1), (0, 0))
                    ).reshape(N, Ch, padw)

    wlt = jnp.transpose(low_w, (0, 2, 3, 1)).reshape(C, 9 * C)
    pw1 = pa_w1.reshape(Cmid, C)
    wlt = jnp.concatenate([wlt, pw1 @ wlt], axis=0).astype(jnp.bfloat16)
    whf = jnp.transpose(high_w[:, :, ::-1, ::-1], (1, 0, 2, 3))
    wht = jnp.transpose(whf, (0, 2, 3, 1)).reshape(C, 9 * Ch)
    wht = wht.astype(jnp.bfloat16)

    def group_w(wt, cin, kx):
        return jnp.concatenate(
            [wt[:, (3 * ky + kx) * cin:(3 * ky + kx + 1) * cin]
             for ky in range(3)], axis=1)

    wlm, wl0, wlp = (group_w(wlt, C, kx) for kx in range(3))
    whm, wh0, whp = (group_w(wht, Ch, kx) for kx in range(3))

    bl = low_b.reshape(C, 1)
    bh = high_b.reshape(C, 1)
    cw1 = ca_w1.reshape(Cmid, C)
    cb1 = ca_b1.reshape(Cmid, 1)
    cw2 = ca_w2.reshape(C, Cmid)
    cb2 = ca_b2.reshape(C, 1)
    pb2m = (pw1 @ low_b.reshape(C, 1)) + pa_b1.reshape(Cmid, 1)
    pw2 = pa_w2.reshape(Cmid, 1)
    pb2 = pa_b2.reshape(1, 1)

    out = pl.pallas_call(
        functools.partial(_meu_kernel, W, HW, 1.0 / float(HW), C),
        out_shape=jax.ShapeDtypeStruct((N, C, HW), jnp.bfloat16),
        grid=(N,),
        in_specs=[
            pl.BlockSpec((1, C, padw), lambda n: (n, 0, 0)),
            pl.BlockSpec((1, Ch, padw), lambda n: (n, 0, 0)),
            pl.BlockSpec((C + Cmid, 3 * C), lambda n: (0, 0)),
            pl.BlockSpec((C + Cmid, 3 * C), lambda n: (0, 0)),
            pl.BlockSpec((C + Cmid, 3 * C), lambda n: (0, 0)),
            pl.BlockSpec((C, 1), lambda n: (0, 0)),
            pl.BlockSpec((C, 3 * Ch), lambda n: (0, 0)),
            pl.BlockSpec((C, 3 * Ch), lambda n: (0, 0)),
            pl.BlockSpec((C, 3 * Ch), lambda n: (0, 0)),
            pl.BlockSpec((C, 1), lambda n: (0, 0)),
            pl.BlockSpec((Cmid, C), lambda n: (0, 0)),
            pl.BlockSpec((Cmid, 1), lambda n: (0, 0)),
            pl.BlockSpec((C, Cmid), lambda n: (0, 0)),
            pl.BlockSpec((C, 1), lambda n: (0, 0)),
            pl.BlockSpec((Cmid, 1), lambda n: (0, 0)),
            pl.BlockSpec((Cmid, 1), lambda n: (0, 0)),
            pl.BlockSpec((1, 1), lambda n: (0, 0)),
        ],
        out_specs=pl.BlockSpec((1, C, HW), lambda n: (n, 0, 0)),
        compiler_params=pltpu.CompilerParams(
            dimension_semantics=("parallel",),
            vmem_limit_bytes=56 * 1024 * 1024),
    )(low_in, hd_in, wlm, wl0, wlp, bl, whm, wh0, whp, bh,
      cw1, cb1, cw2, cb2, pb2m, pw2, pb2)

    return out.astype(low_level.dtype).reshape(N, C, H, W)


# speedup vs baseline: 1.1056x; 1.1056x over previous
import functools

import jax
import jax.numpy as jnp
from jax import lax
from jax.experimental import pallas as pl
from jax.experimental.pallas import tpu as pltpu


def _meu_kernel(W, HW, inv_hw, C,
                low_ref, hd_ref,
                wlm_ref, wl0_ref, wlp_ref, bl_ref,
                whm_ref, wh0_ref, whp_ref, bh_ref,
                cw1_ref, cb1_ref, cw2_ref, cb2_ref,
                pb2m_ref, pw2_ref, pb2_ref,
                o_ref, xs_ref):
    padw = xs_ref.shape[1]
    padl = 2 * W

    xs_ref[:, :padl] = jnp.zeros((xs_ref.shape[0], padl), xs_ref.dtype)
    xs_ref[:, padl + HW:] = jnp.zeros((xs_ref.shape[0], padl), xs_ref.dtype)
    xs_ref[:, padl:padl + HW] = low_ref[0]

    colm = (lax.broadcasted_iota(jnp.int32, (1, padw), 1) - padl) % W
    mL = (colm != W - 1).astype(jnp.bfloat16)
    mR = (colm != 0).astype(jnp.bfloat16)

    def group_stack(v, vl, vr, ox):
        src = vl if ox == -1 else (vr if ox == 1 else v)
        return jnp.concatenate(
            [src[:, padl + oy * W + ox:padl + oy * W + ox + HW]
             for oy in (-1, 0, 1)], axis=0)

    def conv9(v, w_refs):
        vl = v * mL
        vr = v * mR
        return [jnp.dot(w_ref[...], group_stack(v, vl, vr, ox),
                        preferred_element_type=jnp.float32)
                for ox, w_ref in zip((-1, 0, 1), w_refs)]

    e0, e1, e2 = conv9(hd_ref[0], (whm_ref, wh0_ref, whp_ref))
    high = (e0 + e1) + (e2 + bh_ref[...])

    d0, d1, d2 = conv9(xs_ref[...], (wlm_ref, wl0_ref, wlp_ref))
    lw = (d0 + d1) + d2
    low = lw[:C] + bl_ref[...]

    mean = ((jnp.sum(e0, axis=1, keepdims=True)
             + jnp.sum(e1, axis=1, keepdims=True)
             + jnp.sum(e2, axis=1, keepdims=True)) * inv_hw
            + bh_ref[...])
    h1 = jnp.dot(cw1_ref[...], mean, preferred_element_type=jnp.float32)
    h1 = jnp.maximum(h1 + cb1_ref[...], 0.0)
    ca = jax.nn.sigmoid(
        jnp.dot(cw2_ref[...], h1, preferred_element_type=jnp.float32)
        + cb2_ref[...])

    h2 = jnp.maximum(lw[C:] + pb2m_ref[...], 0.0)
    pa = jax.nn.sigmoid(
        jnp.sum(h2 * pw2_ref[...], axis=0, keepdims=True) + pb2_ref[...])

    o_ref[0] = (low * ca + high * pa).astype(o_ref.dtype)


def kernel(low_level, high_level, low_w, low_b, high_w, high_b,
           ca_w1, ca_b1, ca_w2, ca_b2, pa_w1, pa_b1, pa_w2, pa_b2):
    N, C, H, W = low_level.shape
    Ch, H2, W2 = high_level.shape[1], high_level.shape[2], high_level.shape[3]
    HW = H * W
    Cmid = pa_w1.shape[0]
    padw = HW + 4 * W

    low_in = low_level.astype(jnp.bfloat16).reshape(N, C, HW)

    eye = jnp.zeros((2, 2), jnp.bfloat16).at[0, 0].set(jnp.bfloat16(1))
    hd6 = (high_level.astype(jnp.bfloat16)[:, :, :, None, :, None]
           * eye[None, None, None, :, None, :])
    hd_in = jnp.pad(hd6, ((0, 0), (0, 0), (1, 1), (0, 0), (0, 0), (0, 0))
                    ).reshape(N, Ch, padw)

    wlt = jnp.transpose(low_w, (0, 2, 3, 1)).reshape(C, 9 * C)
    pw1 = pa_w1.reshape(Cmid, C)
    wlt = jnp.concatenate([wlt, pw1 @ wlt], axis=0).astype(jnp.bfloat16)
    whf = jnp.transpose(high_w[:, :, ::-1, ::-1], (1, 0, 2, 3))
    wht = jnp.transpose(whf, (0, 2, 3, 1)).reshape(C, 9 * Ch)
    wht = wht.astype(jnp.bfloat16)

    def group_w(wt, cin, kx):
        return jnp.concatenate(
            [wt[:, (3 * ky + kx) * cin:(3 * ky + kx + 1) * cin]
             for ky in range(3)], axis=1)

    wlm, wl0, wlp = (group_w(wlt, C, kx) for kx in range(3))
    whm, wh0, whp = (group_w(wht, Ch, kx) for kx in range(3))

    bl = low_b.reshape(C, 1)
    bh = high_b.reshape(C, 1)
    cw1 = ca_w1.reshape(Cmid, C)
    cb1 = ca_b1.reshape(Cmid, 1)
    cw2 = ca_w2.reshape(C, Cmid)
    cb2 = ca_b2.reshape(C, 1)
    pb2m = (pw1 @ low_b.reshape(C, 1)) + pa_b1.reshape(Cmid, 1)
    pw2 = pa_w2.reshape(Cmid, 1)
    pb2 = pa_b2.reshape(1, 1)

    out = pl.pallas_call(
        functools.partial(_meu_kernel, W, HW, 1.0 / float(HW), C),
        out_shape=jax.ShapeDtypeStruct((N, C, HW), jnp.bfloat16),
        grid=(N,),
        in_specs=[
            pl.BlockSpec((1, C, HW), lambda n: (n, 0, 0)),
            pl.BlockSpec((1, Ch, padw), lambda n: (n, 0, 0)),
            pl.BlockSpec((C + Cmid, 3 * C), lambda n: (0, 0)),
            pl.BlockSpec((C + Cmid, 3 * C), lambda n: (0, 0)),
            pl.BlockSpec((C + Cmid, 3 * C), lambda n: (0, 0)),
            pl.BlockSpec((C, 1), lambda n: (0, 0)),
            pl.BlockSpec((C, 3 * Ch), lambda n: (0, 0)),
            pl.BlockSpec((C, 3 * Ch), lambda n: (0, 0)),
            pl.BlockSpec((C, 3 * Ch), lambda n: (0, 0)),
            pl.BlockSpec((C, 1), lambda n: (0, 0)),
            pl.BlockSpec((Cmid, C), lambda n: (0, 0)),
            pl.BlockSpec((Cmid, 1), lambda n: (0, 0)),
            pl.BlockSpec((C, Cmid), lambda n: (0, 0)),
            pl.BlockSpec((C, 1), lambda n: (0, 0)),
            pl.BlockSpec((Cmid, 1), lambda n: (0, 0)),
            pl.BlockSpec((Cmid, 1), lambda n: (0, 0)),
            pl.BlockSpec((1, 1), lambda n: (0, 0)),
        ],
        out_specs=pl.BlockSpec((1, C, HW), lambda n: (n, 0, 0)),
        scratch_shapes=[pltpu.VMEM((C, padw), jnp.bfloat16)],
        compiler_params=pltpu.CompilerParams(
            dimension_semantics=("parallel",),
            vmem_limit_bytes=56 * 1024 * 1024),
    )(low_in, hd_in, wlm, wl0, wlp, bl, whm, wh0, whp, bh,
      cw1, cb1, cw2, cb2, pb2m, pw2, pb2)

    return out.astype(low_level.dtype).reshape(N, C, H, W)


# speedup vs baseline: 1.1434x; 1.0342x over previous
import functools

import jax
import jax.numpy as jnp
from jax import lax
from jax.experimental import pallas as pl
from jax.experimental.pallas import tpu as pltpu


def _meu_kernel(W, HW, inv_hw, C,
                low_ref, hd_ref,
                wlm_ref, wl0_ref, wlp_ref, bl_ref,
                whm_ref, wh0_ref, whp_ref, bh_ref,
                cw1_ref, cb1_ref, cw2_ref, cb2_ref,
                pb2m_ref, pw2_ref, pb2_ref,
                o_ref, xs_ref, hs_ref):
    padw = xs_ref.shape[1]
    padl = 2 * W

    for ref, src in ((xs_ref, low_ref), (hs_ref, hd_ref)):
        ref[:, :padl] = jnp.zeros((ref.shape[0], padl), ref.dtype)
        ref[:, padl + HW:] = jnp.zeros((ref.shape[0], padl), ref.dtype)
        ref[:, padl:padl + HW] = src[0]

    colm = (lax.broadcasted_iota(jnp.int32, (1, padw), 1) - padl) % W
    mL = (colm != W - 1).astype(jnp.bfloat16)
    mR = (colm != 0).astype(jnp.bfloat16)

    def group_stack(v, vl, vr, ox):
        src = vl if ox == -1 else (vr if ox == 1 else v)
        return jnp.concatenate(
            [src[:, padl + oy * W + ox:padl + oy * W + ox + HW]
             for oy in (-1, 0, 1)], axis=0)

    def conv9(v, w_refs):
        vl = v * mL
        vr = v * mR
        return [jnp.dot(w_ref[...], group_stack(v, vl, vr, ox),
                        preferred_element_type=jnp.float32)
                for ox, w_ref in zip((-1, 0, 1), w_refs)]

    e0, e1, e2 = conv9(hs_ref[...], (whm_ref, wh0_ref, whp_ref))
    high = (e0 + e1) + (e2 + bh_ref[...])

    d0, d1, d2 = conv9(xs_ref[...], (wlm_ref, wl0_ref, wlp_ref))
    lw = (d0 + d1) + d2
    low = lw[:C] + bl_ref[...]

    mean = ((jnp.sum(e0, axis=1, keepdims=True)
             + jnp.sum(e1, axis=1, keepdims=True)
             + jnp.sum(e2, axis=1, keepdims=True)) * inv_hw
            + bh_ref[...])
    h1 = jnp.dot(cw1_ref[...], mean, preferred_element_type=jnp.float32)
    h1 = jnp.maximum(h1 + cb1_ref[...], 0.0)
    ca = jax.nn.sigmoid(
        jnp.dot(cw2_ref[...], h1, preferred_element_type=jnp.float32)
        + cb2_ref[...])

    h2 = jnp.maximum(lw[C:] + pb2m_ref[...], 0.0)
    pa = jax.nn.sigmoid(
        jnp.sum(h2 * pw2_ref[...], axis=0, keepdims=True) + pb2_ref[...])

    o_ref[0] = (low * ca + high * pa).astype(o_ref.dtype)


def kernel(low_level, high_level, low_w, low_b, high_w, high_b,
           ca_w1, ca_b1, ca_w2, ca_b2, pa_w1, pa_b1, pa_w2, pa_b2):
    N, C, H, W = low_level.shape
    Ch, H2, W2 = high_level.shape[1], high_level.shape[2], high_level.shape[3]
    HW = H * W
    Cmid = pa_w1.shape[0]
    padw = HW + 4 * W

    low_in = low_level.astype(jnp.bfloat16).reshape(N, C, HW)

    eye = jnp.zeros((2, 2), jnp.bfloat16).at[0, 0].set(jnp.bfloat16(1))
    hd6 = (high_level.astype(jnp.bfloat16)[:, :, :, None, :, None]
           * eye[None, None, None, :, None, :])
    hd_in = hd6.reshape(N, Ch, HW)

    wlt = jnp.transpose(low_w, (0, 2, 3, 1)).reshape(C, 9 * C)
    pw1 = pa_w1.reshape(Cmid, C)
    wlt = jnp.concatenate([wlt, pw1 @ wlt], axis=0).astype(jnp.bfloat16)
    whf = jnp.transpose(high_w[:, :, ::-1, ::-1], (1, 0, 2, 3))
    wht = jnp.transpose(whf, (0, 2, 3, 1)).reshape(C, 9 * Ch)
    wht = wht.astype(jnp.bfloat16)

    def group_w(wt, cin, kx):
        return jnp.concatenate(
            [wt[:, (3 * ky + kx) * cin:(3 * ky + kx + 1) * cin]
             for ky in range(3)], axis=1)

    wlm, wl0, wlp = (group_w(wlt, C, kx) for kx in range(3))
    whm, wh0, whp = (group_w(wht, Ch, kx) for kx in range(3))

    bl = low_b.reshape(C, 1)
    bh = high_b.reshape(C, 1)
    cw1 = ca_w1.reshape(Cmid, C)
    cb1 = ca_b1.reshape(Cmid, 1)
    cw2 = ca_w2.reshape(C, Cmid)
    cb2 = ca_b2.reshape(C, 1)
    pb2m = (pw1 @ low_b.reshape(C, 1)) + pa_b1.reshape(Cmid, 1)
    pw2 = pa_w2.reshape(Cmid, 1)
    pb2 = pa_b2.reshape(1, 1)

    out = pl.pallas_call(
        functools.partial(_meu_kernel, W, HW, 1.0 / float(HW), C),
        out_shape=jax.ShapeDtypeStruct((N, C, HW), jnp.bfloat16),
        grid=(N,),
        in_specs=[
            pl.BlockSpec((1, C, HW), lambda n: (n, 0, 0)),
            pl.BlockSpec((1, Ch, HW), lambda n: (n, 0, 0)),
            pl.BlockSpec((C + Cmid, 3 * C), lambda n: (0, 0)),
            pl.BlockSpec((C + Cmid, 3 * C), lambda n: (0, 0)),
            pl.BlockSpec((C + Cmid, 3 * C), lambda n: (0, 0)),
            pl.BlockSpec((C, 1), lambda n: (0, 0)),
            pl.BlockSpec((C, 3 * Ch), lambda n: (0, 0)),
            pl.BlockSpec((C, 3 * Ch), lambda n: (0, 0)),
            pl.BlockSpec((C, 3 * Ch), lambda n: (0, 0)),
            pl.BlockSpec((C, 1), lambda n: (0, 0)),
            pl.BlockSpec((Cmid, C), lambda n: (0, 0)),
            pl.BlockSpec((Cmid, 1), lambda n: (0, 0)),
            pl.BlockSpec((C, Cmid), lambda n: (0, 0)),
            pl.BlockSpec((C, 1), lambda n: (0, 0)),
            pl.BlockSpec((Cmid, 1), lambda n: (0, 0)),
            pl.BlockSpec((Cmid, 1), lambda n: (0, 0)),
            pl.BlockSpec((1, 1), lambda n: (0, 0)),
        ],
        out_specs=pl.BlockSpec((1, C, HW), lambda n: (n, 0, 0)),
        scratch_shapes=[
            pltpu.VMEM((C, padw), jnp.bfloat16),
            pltpu.VMEM((Ch, padw), jnp.bfloat16),
        ],
        compiler_params=pltpu.CompilerParams(
            dimension_semantics=("parallel",),
            vmem_limit_bytes=56 * 1024 * 1024),
    )(low_in, hd_in, wlm, wl0, wlp, bl, whm, wh0, whp, bh,
      cw1, cb1, cw2, cb2, pb2m, pw2, pb2)

    return out.astype(low_level.dtype).reshape(N, C, H, W)
